# baseline (speedup 1.0000x reference)
import jax
import jax.numpy as jnp
from jax.experimental import pallas as pl

N_NODES = 10000
D = 256
H = 8
O = 32
ROW_BLK = 1000


def _dense_body(x_ref, wf_ref, bp_ref, h_ref, p_ref):
    x = x_ref[...]
    h_ref[...] = jnp.dot(x, wf_ref[...], preferred_element_type=jnp.float32)
    p_ref[...] = jnp.dot(x, bp_ref[...], preferred_element_type=jnp.float32)


def _dense(x, wf, bp):
    n = x.shape[0]
    return pl.pallas_call(
        _dense_body,
        grid=(n // ROW_BLK,),
        in_specs=[
            pl.BlockSpec((ROW_BLK, D), lambda i: (i, 0)),
            pl.BlockSpec((D, D), lambda i: (0, 0)),
            pl.BlockSpec((D, 32), lambda i: (0, 0)),
        ],
        out_specs=[
            pl.BlockSpec((ROW_BLK, D), lambda i: (i, 0)),
            pl.BlockSpec((ROW_BLK, 32), lambda i: (i, 0)),
        ],
        out_shape=[
            jax.ShapeDtypeStruct((n, D), jnp.float32),
            jax.ShapeDtypeStruct((n, 32), jnp.float32),
        ],
    )(x, wf, bp)


def _post_dense_body(agg_ref, g_ref, b_ref, wf_ref, bp_ref, h_ref, p_ref):
    x = jnp.maximum(agg_ref[...], 0.0)
    mu = jnp.mean(x, axis=-1, keepdims=True)
    var = jnp.mean((x - mu) ** 2, axis=-1, keepdims=True)
    y = (x - mu) / jnp.sqrt(var + 1e-5) * g_ref[...] + b_ref[...]
    h_ref[...] = jnp.dot(y, wf_ref[...], preferred_element_type=jnp.float32)
    p_ref[...] = jnp.dot(y, bp_ref[...], preferred_element_type=jnp.float32)


def _post_dense(agg, gamma, beta, wf, bp):
    n = agg.shape[0]
    return pl.pallas_call(
        _post_dense_body,
        grid=(n // ROW_BLK,),
        in_specs=[
            pl.BlockSpec((ROW_BLK, D), lambda i: (i, 0)),
            pl.BlockSpec((1, D), lambda i: (0, 0)),
            pl.BlockSpec((1, D), lambda i: (0, 0)),
            pl.BlockSpec((D, D), lambda i: (0, 0)),
            pl.BlockSpec((D, 32), lambda i: (0, 0)),
        ],
        out_specs=[
            pl.BlockSpec((ROW_BLK, D), lambda i: (i, 0)),
            pl.BlockSpec((ROW_BLK, 32), lambda i: (i, 0)),
        ],
        out_shape=[
            jax.ShapeDtypeStruct((n, D), jnp.float32),
            jax.ShapeDtypeStruct((n, 32), jnp.float32),
        ],
    )(agg, gamma.reshape(1, D), beta.reshape(1, D), wf, bp)


def _post_final_body(agg_ref, g_ref, b_ref, y_ref):
    x = jnp.maximum(agg_ref[...], 0.0)
    mu = jnp.mean(x, axis=-1, keepdims=True)
    var = jnp.mean((x - mu) ** 2, axis=-1, keepdims=True)
    y_ref[...] = (x - mu) / jnp.sqrt(var + 1e-5) * g_ref[...] + b_ref[...]


def _post_final(agg, gamma, beta):
    n = agg.shape[0]
    return pl.pallas_call(
        _post_final_body,
        grid=(n // ROW_BLK,),
        in_specs=[
            pl.BlockSpec((ROW_BLK, D), lambda i: (i, 0)),
            pl.BlockSpec((1, D), lambda i: (0, 0)),
            pl.BlockSpec((1, D), lambda i: (0, 0)),
        ],
        out_specs=pl.BlockSpec((ROW_BLK, D), lambda i: (i, 0)),
        out_shape=jax.ShapeDtypeStruct((n, D), jnp.float32),
    )(agg, gamma.reshape(1, D), beta.reshape(1, D))


def _leaky(x):
    return jnp.where(x > 0, x, 0.2 * x)


def _edge_softmax_agg(h_flat, p, src, dst):
    n = h_flat.shape[0]
    s_e = p[src, :8]
    d_e = p[dst, 8:16]
    e = _leaky(s_e + d_e)
    m = jax.ops.segment_max(e, dst, num_segments=n)
    m = jnp.where(jnp.isfinite(m), m, 0.0)
    pexp = jnp.exp(e - m[dst])
    denom = jax.ops.segment_sum(pexp, dst, num_segments=n)
    alpha = pexp / (denom[dst] + 1e-8)
    msg = h_flat[src].reshape(-1, H, O) * alpha[:, :, None]
    return jax.ops.segment_sum(msg.reshape(-1, H * O), dst, num_segments=n)


def _prep_weights(W, a_src, a_dst):
    wf = jnp.transpose(W, (1, 0, 2)).reshape(D, H * O)
    bs = jnp.einsum('hio,ho->ih', W, a_src[..., 0])
    bd = jnp.einsum('hio,ho->ih', W, a_dst[..., 0])
    bp = jnp.concatenate([bs, bd, bd, bs], axis=1)
    return wf, bp


def kernel(emb, W0, a_src0, a_dst0, W1, a_src1, a_dst1, gamma, beta, entity_ids, edge_index):
    src = edge_index[0]
    dst = edge_index[1]
    wf0, bp0 = _prep_weights(W0, a_src0, a_dst0)
    wf1, bp1 = _prep_weights(W1, a_src1, a_dst1)

    h1, p1 = _dense(emb, wf0, bp0)
    agg1 = _edge_softmax_agg(h1, p1, src, dst)
    h2, p2 = _post_dense(agg1, gamma, beta, wf1, bp1)
    agg2 = _edge_softmax_agg(h2, p2, src, dst)
    y = _post_final(agg2, gamma, beta)
    return y[entity_ids]


# speedup vs baseline: 37.3402x; 37.3402x over previous
import jax
import jax.numpy as jnp
from jax.experimental import pallas as pl

N_NODES = 10000
D = 256
H = 8
O = 32
ROW_BLK = 1000


def _dense_body(x_ref, wf_ref, bp_ref, h_ref, p_ref):
    x = x_ref[...]
    h_ref[...] = jnp.dot(x, wf_ref[...], preferred_element_type=jnp.float32)
    p_ref[...] = jnp.dot(x, bp_ref[...], preferred_element_type=jnp.float32)


def _dense(x, wf, bp):
    n = x.shape[0]
    return pl.pallas_call(
        _dense_body,
        grid=(n // ROW_BLK,),
        in_specs=[
            pl.BlockSpec((ROW_BLK, D), lambda i: (i, 0)),
            pl.BlockSpec((D, D), lambda i: (0, 0)),
            pl.BlockSpec((D, 32), lambda i: (0, 0)),
        ],
        out_specs=[
            pl.BlockSpec((ROW_BLK, D), lambda i: (i, 0)),
            pl.BlockSpec((ROW_BLK, 32), lambda i: (i, 0)),
        ],
        out_shape=[
            jax.ShapeDtypeStruct((n, D), jnp.float32),
            jax.ShapeDtypeStruct((n, 32), jnp.float32),
        ],
    )(x, wf, bp)


def _post_dense_body(agg_ref, g_ref, b_ref, wf_ref, bp_ref, h_ref, p_ref):
    x = jnp.maximum(agg_ref[...], 0.0)
    mu = jnp.mean(x, axis=-1, keepdims=True)
    var = jnp.mean((x - mu) ** 2, axis=-1, keepdims=True)
    y = (x - mu) / jnp.sqrt(var + 1e-5) * g_ref[...] + b_ref[...]
    h_ref[...] = jnp.dot(y, wf_ref[...], preferred_element_type=jnp.float32)
    p_ref[...] = jnp.dot(y, bp_ref[...], preferred_element_type=jnp.float32)


def _post_dense(agg, gamma, beta, wf, bp):
    n = agg.shape[0]
    return pl.pallas_call(
        _post_dense_body,
        grid=(n // ROW_BLK,),
        in_specs=[
            pl.BlockSpec((ROW_BLK, D), lambda i: (i, 0)),
            pl.BlockSpec((1, D), lambda i: (0, 0)),
            pl.BlockSpec((1, D), lambda i: (0, 0)),
            pl.BlockSpec((D, D), lambda i: (0, 0)),
            pl.BlockSpec((D, 32), lambda i: (0, 0)),
        ],
        out_specs=[
            pl.BlockSpec((ROW_BLK, D), lambda i: (i, 0)),
            pl.BlockSpec((ROW_BLK, 32), lambda i: (i, 0)),
        ],
        out_shape=[
            jax.ShapeDtypeStruct((n, D), jnp.float32),
            jax.ShapeDtypeStruct((n, 32), jnp.float32),
        ],
    )(agg, gamma.reshape(1, D), beta.reshape(1, D), wf, bp)


def _post_final_body(agg_ref, g_ref, b_ref, y_ref):
    x = jnp.maximum(agg_ref[...], 0.0)
    mu = jnp.mean(x, axis=-1, keepdims=True)
    var = jnp.mean((x - mu) ** 2, axis=-1, keepdims=True)
    y_ref[...] = (x - mu) / jnp.sqrt(var + 1e-5) * g_ref[...] + b_ref[...]


def _post_final(agg, gamma, beta):
    n = agg.shape[0]
    return pl.pallas_call(
        _post_final_body,
        grid=(n // ROW_BLK,),
        in_specs=[
            pl.BlockSpec((ROW_BLK, D), lambda i: (i, 0)),
            pl.BlockSpec((1, D), lambda i: (0, 0)),
            pl.BlockSpec((1, D), lambda i: (0, 0)),
        ],
        out_specs=pl.BlockSpec((ROW_BLK, D), lambda i: (i, 0)),
        out_shape=jax.ShapeDtypeStruct((n, D), jnp.float32),
    )(agg, gamma.reshape(1, D), beta.reshape(1, D))


def _leaky(x):
    return jnp.where(x > 0, x, 0.2 * x)


def _edge_softmax_agg(h_flat, p, src, dst):
    n = h_flat.shape[0]
    s_e = p[:, :8][src]
    d_e = p[:, 8:16][dst]
    e = _leaky(s_e + d_e)
    m = jax.ops.segment_max(e, dst, num_segments=n)
    m = jnp.where(jnp.isfinite(m), m, 0.0)
    pexp = jnp.exp(e - m[dst])
    denom = jax.ops.segment_sum(pexp, dst, num_segments=n)
    alpha = pexp / (denom[dst] + 1e-8)
    msg = h_flat[src].reshape(-1, H, O) * alpha[:, :, None]
    return jax.ops.segment_sum(msg.reshape(-1, H * O), dst, num_segments=n)


def _prep_weights(W, a_src, a_dst):
    wf = jnp.transpose(W, (1, 0, 2)).reshape(D, H * O)
    bs = jnp.einsum('hio,ho->ih', W, a_src[..., 0])
    bd = jnp.einsum('hio,ho->ih', W, a_dst[..., 0])
    bp = jnp.concatenate([bs, bd, bd, bs], axis=1)
    return wf, bp


def kernel(emb, W0, a_src0, a_dst0, W1, a_src1, a_dst1, gamma, beta, entity_ids, edge_index):
    src = edge_index[0]
    dst = edge_index[1]
    wf0, bp0 = _prep_weights(W0, a_src0, a_dst0)
    wf1, bp1 = _prep_weights(W1, a_src1, a_dst1)

    h1, p1 = _dense(emb, wf0, bp0)
    agg1 = _edge_softmax_agg(h1, p1, src, dst)
    h2, p2 = _post_dense(agg1, gamma, beta, wf1, bp1)
    agg2 = _edge_softmax_agg(h2, p2, src, dst)
    y = _post_final(agg2, gamma, beta)
    return y[entity_ids]


# speedup vs baseline: 43.3851x; 1.1619x over previous
import jax
import jax.numpy as jnp
from jax import lax
from jax.experimental import pallas as pl
from jax.experimental.pallas import tpu as pltpu
from jax.experimental.pallas import tpu_sc as plsc

N_NODES = 10000
D = 256
H = 8
O = 32
ROW_BLK = 1000

EC = 64
EP = 163840
CH_PER_TILE = EP // (16 * EC)
KBUF = 4
NP = 10240
NROWS_PER_TILE = NP // 16


def _dense_body(x_ref, wf_ref, bp_ref, h_ref, p_ref):
    x = x_ref[...]
    ht = jnp.dot(x, wf_ref[...], preferred_element_type=jnp.float32)
    h_ref[0] = ht[:, :128]
    h_ref[1] = ht[:, 128:]
    p_ref[...] = jnp.dot(x, bp_ref[...], preferred_element_type=jnp.float32)


def _dense(x, wf, bp):
    n = x.shape[0]
    return pl.pallas_call(
        _dense_body,
        grid=(n // ROW_BLK,),
        in_specs=[
            pl.BlockSpec((ROW_BLK, D), lambda i: (i, 0)),
            pl.BlockSpec((D, D), lambda i: (0, 0)),
            pl.BlockSpec((D, 32), lambda i: (0, 0)),
        ],
        out_specs=[
            pl.BlockSpec((2, ROW_BLK, 128), lambda i: (0, i, 0)),
            pl.BlockSpec((ROW_BLK, 32), lambda i: (i, 0)),
        ],
        out_shape=[
            jax.ShapeDtypeStruct((2, n, 128), jnp.float32),
            jax.ShapeDtypeStruct((n, 32), jnp.float32),
        ],
    )(x, wf, bp)


def _post_dense_body(agg_ref, g_ref, b_ref, wf_ref, bp_ref, h_ref, p_ref):
    x = jnp.maximum(jnp.concatenate([agg_ref[0], agg_ref[1]], axis=1), 0.0)
    mu = jnp.mean(x, axis=-1, keepdims=True)
    var = jnp.mean((x - mu) ** 2, axis=-1, keepdims=True)
    y = (x - mu) / jnp.sqrt(var + 1e-5) * g_ref[...] + b_ref[...]
    ht = jnp.dot(y, wf_ref[...], preferred_element_type=jnp.float32)
    h_ref[0] = ht[:, :128]
    h_ref[1] = ht[:, 128:]
    p_ref[...] = jnp.dot(y, bp_ref[...], preferred_element_type=jnp.float32)


def _post_dense(agg, gamma, beta, wf, bp):
    n = agg.shape[1]
    return pl.pallas_call(
        _post_dense_body,
        grid=(n // ROW_BLK,),
        in_specs=[
            pl.BlockSpec((2, ROW_BLK, 128), lambda i: (0, i, 0)),
            pl.BlockSpec((1, D), lambda i: (0, 0)),
            pl.BlockSpec((1, D), lambda i: (0, 0)),
            pl.BlockSpec((D, D), lambda i: (0, 0)),
            pl.BlockSpec((D, 32), lambda i: (0, 0)),
        ],
        out_specs=[
            pl.BlockSpec((2, ROW_BLK, 128), lambda i: (0, i, 0)),
            pl.BlockSpec((ROW_BLK, 32), lambda i: (i, 0)),
        ],
        out_shape=[
            jax.ShapeDtypeStruct((2, n, 128), jnp.float32),
            jax.ShapeDtypeStruct((n, 32), jnp.float32),
        ],
    )(agg, gamma.reshape(1, D), beta.reshape(1, D), wf, bp)


def _post_final_body(agg_ref, g_ref, b_ref, y_ref):
    x = jnp.maximum(jnp.concatenate([agg_ref[0], agg_ref[1]], axis=1), 0.0)
    mu = jnp.mean(x, axis=-1, keepdims=True)
    var = jnp.mean((x - mu) ** 2, axis=-1, keepdims=True)
    y_ref[...] = (x - mu) / jnp.sqrt(var + 1e-5) * g_ref[...] + b_ref[...]


def _post_final(agg, gamma, beta):
    n = agg.shape[1]
    return pl.pallas_call(
        _post_final_body,
        grid=(n // ROW_BLK,),
        in_specs=[
            pl.BlockSpec((2, ROW_BLK, 128), lambda i: (0, i, 0)),
            pl.BlockSpec((1, D), lambda i: (0, 0)),
            pl.BlockSpec((1, D), lambda i: (0, 0)),
        ],
        out_specs=pl.BlockSpec((ROW_BLK, D), lambda i: (i, 0)),
        out_shape=jax.ShapeDtypeStruct((n, D), jnp.float32),
    )(agg, gamma.reshape(1, D), beta.reshape(1, D))


def _leaky(x):
    return jnp.where(x > 0, x, 0.2 * x)


def _edge_alpha(p, src, dst):
    n = p.shape[0]
    s_e = p[:, :8][src]
    d_e = p[:, 8:16][dst]
    e = _leaky(s_e + d_e)
    m = jax.ops.segment_max(e, dst, num_segments=n)
    m = jnp.where(jnp.isfinite(m), m, 0.0)
    pexp = jnp.exp(e - m[dst])
    denom = jax.ops.segment_sum(pexp, dst, num_segments=n)
    return pexp / (denom[dst] + 1e-8)


def _sc_agg_body(h2n, srcp2, dstp, alphap, zeros, out, acc,
                 s0, s1, s2, s3, d0, d1, d2, d3, alpha_v,
                 r0, r1, r2, r3, m0, m1, m2, m3):
    c = lax.axis_index("c")
    s = lax.axis_index("s")
    srcs = [s0, s1, s2, s3]
    dsts = [d0, d1, d2, d3]
    rows = [r0, r1, r2, r3]
    sems = [m0, m1, m2, m3]

    pltpu.sync_copy(zeros, acc.at[pl.ds(s * NROWS_PER_TILE, NROWS_PER_TILE)])
    plsc.subcore_barrier()

    tilebase = s * CH_PER_TILE

    def fire(g, b):
        base = (tilebase + g) * EC
        pltpu.sync_copy(srcp2.at[c, pl.ds(base, EC)], srcs[b])
        pltpu.async_copy(h2n.at[srcs[b]], rows[b], sems[b])

    for b in range(KBUF):
        fire(b, b)

    def outer(gg, carry):
        for b in range(KBUF):
            g = gg * KBUF + b
            base = (tilebase + g) * EC
            pltpu.sync_copy(dstp.at[pl.ds(base, EC)], dsts[b])
            pltpu.sync_copy(alphap.at[c, pl.ds(base * 4, EC * 4)], alpha_v)
            pltpu.make_async_copy(h2n.at[pl.ds(0, EC)], rows[b], sems[b]).wait()

            def mul_body(q, carry2, b=b):
                av16 = alpha_v[pl.ds(q * 16, 16)]
                for r in range(4):
                    e = q * 4 + r
                    for jj in range(4):
                        av = jnp.full((16,), av16[r * 4 + jj], jnp.float32)
                        rows[b][e, pl.ds(jj * 32, 16)] = (
                            rows[b][e, pl.ds(jj * 32, 16)] * av)
                        rows[b][e, pl.ds(jj * 32 + 16, 16)] = (
                            rows[b][e, pl.ds(jj * 32 + 16, 16)] * av)
                return carry2

            lax.fori_loop(0, EC // 4, mul_body, 0)
            pltpu.sync_copy(rows[b], acc.at[dsts[b]], add=True)

            @pl.when(gg < CH_PER_TILE // KBUF - 1)
            def _fire_next(g=g, b=b):
                fire(g + KBUF, b)
        return carry

    lax.fori_loop(0, CH_PER_TILE // KBUF, outer, 0)
    plsc.subcore_barrier()
    pltpu.sync_copy(acc.at[pl.ds(s * NROWS_PER_TILE, NROWS_PER_TILE)],
                    out.at[c, pl.ds(s * NROWS_PER_TILE, NROWS_PER_TILE)])


def _sc_agg(h2, alpha, srcp2, dstp, zeros):
    e_real = alpha.shape[0]
    h2n = h2.reshape(2 * N_NODES, 128)
    ap = jnp.concatenate(
        [alpha, jnp.zeros((EP - e_real, H), jnp.float32)])
    alphap = jnp.stack([ap[:, :4], ap[:, 4:]]).reshape(2, EP * 4)
    mesh = plsc.VectorSubcoreMesh(core_axis_name="c", subcore_axis_name="s")
    out = pl.kernel(
        _sc_agg_body,
        out_type=jax.ShapeDtypeStruct((2, NP, 128), jnp.float32),
        mesh=mesh,
        scratch_types=(
            [pltpu.VMEM_SHARED((NP, 128), jnp.float32)]
            + [pltpu.VMEM((EC,), jnp.int32) for _ in range(2 * KBUF)]
            + [pltpu.VMEM((EC * 4,), jnp.float32)]
            + [pltpu.VMEM((EC, 128), jnp.float32) for _ in range(KBUF)]
            + [pltpu.SemaphoreType.DMA for _ in range(KBUF)]
        ),
    )(h2n, srcp2, dstp, alphap, zeros)
    return out[:, :N_NODES, :]


def _prep_weights(W, a_src, a_dst):
    wf = jnp.transpose(W, (1, 0, 2)).reshape(D, H * O)
    bs = jnp.einsum('hio,ho->ih', W, a_src[..., 0])
    bd = jnp.einsum('hio,ho->ih', W, a_dst[..., 0])
    bp = jnp.concatenate([bs, bd, bd, bs], axis=1)
    return wf, bp


def kernel(emb, W0, a_src0, a_dst0, W1, a_src1, a_dst1, gamma, beta, entity_ids, edge_index):
    src = edge_index[0]
    dst = edge_index[1]
    wf0, bp0 = _prep_weights(W0, a_src0, a_dst0)
    wf1, bp1 = _prep_weights(W1, a_src1, a_dst1)

    e_real = src.shape[0]
    pad = jnp.zeros((EP - e_real,), jnp.int32)
    srcp = jnp.concatenate([src, pad])
    srcp2 = jnp.stack([srcp, srcp + N_NODES])
    dstp = jnp.concatenate([dst, pad])
    zeros = jnp.zeros((NROWS_PER_TILE, 128), jnp.float32)

    h1, p1 = _dense(emb, wf0, bp0)
    alpha1 = _edge_alpha(p1, src, dst)
    agg1 = _sc_agg(h1, alpha1, srcp2, dstp, zeros)
    h2, p2 = _post_dense(agg1, gamma, beta, wf1, bp1)
    alpha2 = _edge_alpha(p2, src, dst)
    agg2 = _sc_agg(h2, alpha2, srcp2, dstp, zeros)
    y = _post_final(agg2, gamma, beta)
    return y[entity_ids]
